# baseline (speedup 1.0000x reference)
import functools

import jax
import jax.numpy as jnp
from jax import lax
from jax.experimental import pallas as pl
from jax.experimental.pallas import tpu as pltpu
from jax.experimental.pallas import tpu_sc as plsc

B = 16384
D = 64
H = 4 * D
W128 = 2 * D
NC, NS = 2, 16
NW = NC * NS
BPW = B // NW

_mesh = plsc.VectorSubcoreMesh(
    core_axis_name="c", subcore_axis_name="s", num_cores=NC, num_subcores=NS
)


def _worker_gather(tbl, idx_hbm, out_hbm, idx_v, rows_v, sem, base):
    pltpu.sync_copy(idx_hbm.at[pl.ds(base, BPW)], idx_v)

    def body(g, _):
        vl = idx_v[pl.ds(g * 16, 16)]
        for j in range(16):
            pltpu.async_copy(tbl.at[vl[j]], rows_v.at[g * 16 + j], sem)
        return ()

    lax.fori_loop(0, BPW // 16, body, ())
    pltpu.make_async_copy(out_hbm.at[pl.ds(base, BPW)], rows_v, sem).wait()
    pltpu.sync_copy(rows_v, out_hbm.at[pl.ds(base, BPW)])


@functools.partial(
    pl.kernel,
    out_type=(
        jax.ShapeDtypeStruct((B, W128), jnp.float32),
        jax.ShapeDtypeStruct((B, W128), jnp.float32),
        jax.ShapeDtypeStruct((B, W128), jnp.float32),
    ),
    mesh=_mesh,
    scratch_types=[
        pltpu.VMEM((BPW,), jnp.int32),
        pltpu.VMEM((BPW, W128), jnp.float32),
        pltpu.SemaphoreType.DMA,
    ],
)
def _sc_gather_small(tgP, talP, tcP, ig2, ial2, ic2, og, oal, oc,
                     idx_v, rows_v, sem):
    wid = lax.axis_index("s") * NC + lax.axis_index("c")
    base = wid * BPW
    for tbl, idx_hbm, out_hbm in ((tgP, ig2, og), (talP, ial2, oal),
                                  (tcP, ic2, oc)):
        _worker_gather(tbl, idx_hbm, out_hbm, idx_v, rows_v, sem, base)


@functools.partial(
    pl.kernel,
    out_type=jax.ShapeDtypeStruct((B, W128), jnp.float32),
    mesh=_mesh,
    scratch_types=[
        pltpu.VMEM((BPW,), jnp.int32),
        pltpu.VMEM((BPW, W128), jnp.float32),
        pltpu.SemaphoreType.DMA,
    ],
)
def _sc_gather_artist(taP, ia2, oa, idx_v, rows_v, sem):
    wid = lax.axis_index("s") * NC + lax.axis_index("c")
    base = wid * BPW
    _worker_gather(taP, ia2, oa, idx_v, rows_v, sem, base)


TB = 2048


def _mm_body(xg, xa, xal, xc, sg, sa, sal, sc_, w_ref, b_ref, o_ref):
    parts = []
    for x_ref, s_ref in ((xg, sg), (xa, sa), (xal, sal), (xc, sc_)):
        x = x_ref[...]
        s = s_ref[...]
        parts.append(x[:, :D] * (1.0 - s) + x[:, D:] * s)
    xcat = jnp.concatenate(parts, axis=1)
    acc = lax.dot_general(xcat, w_ref[...], (((1,), (1,)), ((), ())),
                          preferred_element_type=jnp.float32)
    o_ref[...] = acc + b_ref[...]


_mm = pl.pallas_call(
    _mm_body,
    grid=(B // TB,),
    in_specs=[
        pl.BlockSpec((TB, W128), lambda i: (i, 0)),
        pl.BlockSpec((TB, W128), lambda i: (i, 0)),
        pl.BlockSpec((TB, W128), lambda i: (i, 0)),
        pl.BlockSpec((TB, W128), lambda i: (i, 0)),
        pl.BlockSpec((TB, 1), lambda i: (i, 0)),
        pl.BlockSpec((TB, 1), lambda i: (i, 0)),
        pl.BlockSpec((TB, 1), lambda i: (i, 0)),
        pl.BlockSpec((TB, 1), lambda i: (i, 0)),
        pl.BlockSpec((H, H), lambda i: (0, 0)),
        pl.BlockSpec((1, H), lambda i: (0, 0)),
    ],
    out_specs=pl.BlockSpec((TB, H), lambda i: (i, 0)),
    out_shape=jax.ShapeDtypeStruct((B, H), jnp.float32),
    compiler_params=pltpu.CompilerParams(
        dimension_semantics=("parallel",),
    ),
)


def kernel(emb_genre, emb_artist, emb_album, emb_country, W, b,
           idx_genre, idx_artist, idx_album, idx_country):
    def pack2(t):
        return jnp.concatenate([t[0::2], t[1::2]], axis=1)

    tgP = pack2(emb_genre)
    taP = pack2(emb_artist)
    talP = pack2(emb_album)
    tcP = pack2(emb_country)

    def prep(idx):
        idx = idx.astype(jnp.int32)
        return idx >> 1, (idx & 1).astype(jnp.float32).reshape(B, 1)

    ig2, sg = prep(idx_genre)
    ia2, sa = prep(idx_artist)
    ial2, sal = prep(idx_album)
    ic2, sc_ = prep(idx_country)

    g_w, al_w, c_w = _sc_gather_small(tgP, talP, tcP, ig2, ial2, ic2)
    a_w = _sc_gather_artist(taP, ia2)
    return _mm(g_w, a_w, al_w, c_w, sg, sa, sal, sc_, W, b.reshape(1, H))


# speedup vs baseline: 22.1741x; 22.1741x over previous
import functools

import jax
import jax.numpy as jnp
from jax import lax
from jax.experimental import pallas as pl
from jax.experimental.pallas import tpu as pltpu
from jax.experimental.pallas import tpu_sc as plsc

B = 16384
D = 64
H = 4 * D
NC, NS = 2, 16
NW = NC * NS
BPW = B // NW

_mesh = plsc.VectorSubcoreMesh(
    core_axis_name="c", subcore_axis_name="s", num_cores=NC, num_subcores=NS
)


def _worker_gather(tbl, idx_hbm, out_hbm, idx_v, rows_v, sem, base):
    pltpu.sync_copy(idx_hbm.at[pl.ds(base, BPW)], idx_v)

    def body(g, _):
        vl = idx_v[pl.ds(g * 16, 16)]
        for j in range(16):
            pltpu.async_copy(tbl.at[vl[j]], rows_v.at[g * 16 + j], sem)
        return ()

    lax.fori_loop(0, BPW // 16, body, ())
    pltpu.make_async_copy(out_hbm.at[pl.ds(base, BPW)], rows_v, sem).wait()
    pltpu.sync_copy(rows_v, out_hbm.at[pl.ds(base, BPW)])


@functools.partial(
    pl.kernel,
    out_type=(
        jax.ShapeDtypeStruct((B, D), jnp.float32),
        jax.ShapeDtypeStruct((B, D), jnp.float32),
        jax.ShapeDtypeStruct((B, D), jnp.float32),
    ),
    mesh=_mesh,
    scratch_types=[
        pltpu.VMEM((BPW,), jnp.int32),
        pltpu.VMEM((BPW, D), jnp.float32),
        pltpu.SemaphoreType.DMA,
    ],
)
def _sc_gather_small(tg, tal, tc_, ig, ial, ic, og, oal, oc,
                     idx_v, rows_v, sem):
    wid = lax.axis_index("s") * NC + lax.axis_index("c")
    base = wid * BPW
    for tbl, idx_hbm, out_hbm in ((tg, ig, og), (tal, ial, oal),
                                  (tc_, ic, oc)):
        _worker_gather(tbl, idx_hbm, out_hbm, idx_v, rows_v, sem, base)


@functools.partial(
    pl.kernel,
    out_type=jax.ShapeDtypeStruct((B, D), jnp.float32),
    mesh=_mesh,
    scratch_types=[
        pltpu.VMEM((BPW,), jnp.int32),
        pltpu.VMEM((BPW, D), jnp.float32),
        pltpu.SemaphoreType.DMA,
    ],
)
def _sc_gather_artist(ta, ia, oa, idx_v, rows_v, sem):
    wid = lax.axis_index("s") * NC + lax.axis_index("c")
    base = wid * BPW
    _worker_gather(ta, ia, oa, idx_v, rows_v, sem, base)


TB = 2048


def _mm_body(e0, e1, e2, e3, w_ref, b_ref, o_ref):
    x = jnp.concatenate([e0[...], e1[...], e2[...], e3[...]], axis=1)
    acc = lax.dot_general(x, w_ref[...], (((1,), (1,)), ((), ())),
                          preferred_element_type=jnp.float32)
    o_ref[...] = acc + b_ref[...]


_mm = pl.pallas_call(
    _mm_body,
    grid=(B // TB,),
    in_specs=[
        pl.BlockSpec((TB, D), lambda i: (i, 0)),
        pl.BlockSpec((TB, D), lambda i: (i, 0)),
        pl.BlockSpec((TB, D), lambda i: (i, 0)),
        pl.BlockSpec((TB, D), lambda i: (i, 0)),
        pl.BlockSpec((H, H), lambda i: (0, 0)),
        pl.BlockSpec((1, H), lambda i: (0, 0)),
    ],
    out_specs=pl.BlockSpec((TB, H), lambda i: (i, 0)),
    out_shape=jax.ShapeDtypeStruct((B, H), jnp.float32),
    compiler_params=pltpu.CompilerParams(
        dimension_semantics=("parallel",),
    ),
)


def kernel(emb_genre, emb_artist, emb_album, emb_country, W, b,
           idx_genre, idx_artist, idx_album, idx_country):
    e_g, e_al, e_c = _sc_gather_small(
        emb_genre, emb_album, emb_country,
        idx_genre.astype(jnp.int32), idx_album.astype(jnp.int32),
        idx_country.astype(jnp.int32),
    )
    e_a = _sc_gather_artist(emb_artist, idx_artist.astype(jnp.int32))
    return _mm(e_g, e_a, e_al, e_c, W, b.reshape(1, H))


# speedup vs baseline: 22.9459x; 1.0348x over previous
import functools

import jax
import jax.numpy as jnp
from jax import lax
from jax.experimental import pallas as pl
from jax.experimental.pallas import tpu as pltpu
from jax.experimental.pallas import tpu_sc as plsc

B = 16384
D = 64
H = 4 * D
NC, NS = 2, 16
NW = NC * NS
BPW = B // NW

_mesh = plsc.VectorSubcoreMesh(
    core_axis_name="c", subcore_axis_name="s", num_cores=NC, num_subcores=NS
)


@functools.partial(
    pl.kernel,
    out_type=(
        jax.ShapeDtypeStruct((B, 2 * D), jnp.float32),
        jax.ShapeDtypeStruct((B, 2 * D), jnp.float32),
    ),
    mesh=_mesh,
    scratch_types=[
        pltpu.VMEM((BPW,), jnp.int32),
        pltpu.VMEM((BPW,), jnp.int32),
        pltpu.VMEM((BPW, 2 * D), jnp.float32),
        pltpu.SemaphoreType.DMA,
    ],
)
def _sc_gather(tg, ta, tal, tc_, ig, ia, ial, ic, out01, out23,
               idx_l, idx_r, rows_v, sem):
    wid = lax.axis_index("s") * NC + lax.axis_index("c")
    base = wid * BPW
    for tbl_l, idx_hbm_l, tbl_r, idx_hbm_r, out_hbm in (
        (tg, ig, ta, ia, out01),
        (tal, ial, tc_, ic, out23),
    ):
        pltpu.sync_copy(idx_hbm_l.at[pl.ds(base, BPW)], idx_l)
        pltpu.sync_copy(idx_hbm_r.at[pl.ds(base, BPW)], idx_r)

        def body(g, _, tbl_l=tbl_l, tbl_r=tbl_r):
            i0 = g * 16
            vl = idx_l[pl.ds(i0, 16)]
            vr = idx_r[pl.ds(i0, 16)]
            for j in range(16):
                pltpu.async_copy(
                    tbl_l.at[vl[j]], rows_v.at[i0 + j, pl.ds(0, D)], sem)
                pltpu.async_copy(
                    tbl_r.at[vr[j]], rows_v.at[i0 + j, pl.ds(D, D)], sem)
            return ()

        lax.fori_loop(0, BPW // 16, body, ())
        pltpu.make_async_copy(
            out_hbm.at[pl.ds(base, BPW)], rows_v, sem
        ).wait()
        pltpu.sync_copy(rows_v, out_hbm.at[pl.ds(base, BPW)])


TB = 2048


def _mm_body(x01, x23, w_ref, b_ref, o_ref):
    x = jnp.concatenate([x01[...], x23[...]], axis=1)
    acc = lax.dot_general(x, w_ref[...], (((1,), (1,)), ((), ())),
                          preferred_element_type=jnp.float32)
    o_ref[...] = acc + b_ref[...]


_mm = pl.pallas_call(
    _mm_body,
    grid=(B // TB,),
    in_specs=[
        pl.BlockSpec((TB, 2 * D), lambda i: (i, 0)),
        pl.BlockSpec((TB, 2 * D), lambda i: (i, 0)),
        pl.BlockSpec((H, H), lambda i: (0, 0)),
        pl.BlockSpec((1, H), lambda i: (0, 0)),
    ],
    out_specs=pl.BlockSpec((TB, H), lambda i: (i, 0)),
    out_shape=jax.ShapeDtypeStruct((B, H), jnp.float32),
    compiler_params=pltpu.CompilerParams(
        dimension_semantics=("parallel",),
    ),
)


def kernel(emb_genre, emb_artist, emb_album, emb_country, W, b,
           idx_genre, idx_artist, idx_album, idx_country):
    x01, x23 = _sc_gather(
        emb_genre, emb_artist, emb_album, emb_country,
        idx_genre.astype(jnp.int32), idx_artist.astype(jnp.int32),
        idx_album.astype(jnp.int32), idx_country.astype(jnp.int32),
    )
    return _mm(x01, x23, W, b.reshape(1, H))
